# baseline (speedup 1.0000x reference)
import functools

import jax
import jax.numpy as jnp
from jax import lax
from jax.experimental import pallas as pl
from jax.experimental.pallas import tpu as pltpu
from jax.experimental.pallas import tpu_sc as plsc

NUM_WORKERS = 32


@jax.jit
def _gather_lmajor(idx, table):
    b, l_dim = idx.shape
    v, d = table.shape
    bpw = b // NUM_WORKERS

    mesh = plsc.VectorSubcoreMesh(core_axis_name="c", subcore_axis_name="s")

    @functools.partial(
        pl.kernel,
        out_type=jax.ShapeDtypeStruct((l_dim * b, d), jnp.float32),
        mesh=mesh,
        scratch_types=[
            pltpu.VMEM((bpw, l_dim), jnp.int32),
            pltpu.VMEM((2, bpw), jnp.int32),
            pltpu.VMEM((2, bpw, d), jnp.float32),
            pltpu.SemaphoreType.DMA,
            pltpu.SemaphoreType.DMA,
            pltpu.SemaphoreType.DMA,
        ],
        compiler_params=pltpu.CompilerParams(
            use_tc_tiling_on_sc=False, needs_layout_passes=False
        ),
    )
    def k(idx_hbm, table_hbm, out_hbm, idx_v, idxcol, bufg, gsem, osem0, osem1):
        wid = lax.axis_index("s") * 2 + lax.axis_index("c")
        b0 = wid * bpw
        pltpu.sync_copy(idx_hbm.at[pl.ds(b0, bpw)], idx_v)
        lanes = lax.iota(jnp.int32, 16)
        osems = (osem0, osem1)

        def build_idxcol(lidx, sel):
            lvec = jnp.full((16,), lidx, jnp.int32)

            def bg(g, c):
                tau = lanes + g * 16
                beta = tau // 2 + (tau % 2) * 64
                idxcol[sel, pl.ds(g * 16, 16)] = plsc.load_gather(idx_v, [beta, lvec])
                return c

            lax.fori_loop(0, bpw // 16, bg, 0)

        def fire_gather(sel):
            pltpu.async_copy(table_hbm.at[idxcol.at[sel]], bufg.at[sel], gsem)

        def wait_gather(sel):
            pltpu.make_async_copy(
                table_hbm.at[idxcol.at[sel]], bufg.at[sel], gsem
            ).wait()

        def out_slice(lidx):
            return out_hbm.at[pl.ds(lidx * b + b0, bpw)]

        def fire_out(lidx, sel):
            pltpu.async_copy(bufg.at[sel], out_slice(lidx), osems[sel])

        def wait_out(lidx, sel):
            pltpu.make_async_copy(bufg.at[sel], out_slice(lidx), osems[sel]).wait()

        def step(lidx, sel, fire_next, wait_o):
            if fire_next:
                build_idxcol(lidx + 1, 1 - sel)
                fire_gather(1 - sel)
            wait_gather(sel)
            if wait_o:
                wait_out(lidx - 2, sel)
            fire_out(lidx, sel)

        build_idxcol(0, 0)
        fire_gather(0)
        step(0, 0, True, False)
        step(1, 1, True, False)

        def body(i, c):
            step(2 * i + 2, 0, True, True)
            step(2 * i + 3, 1, True, True)
            return c

        lax.fori_loop(0, (l_dim - 4) // 2, body, 0)
        step(l_dim - 2, 0, True, True)
        step(l_dim - 1, 1, False, True)
        wait_out(l_dim - 2, 0)
        wait_out(l_dim - 1, 1)

    return k(idx, table)


def _retile_tc(g3, l_dim, n_bi):
    bi_per = 16

    def tr(x_ref, o_ref):
        ident = jnp.eye(128, dtype=jnp.float32)
        x64 = x_ref[...].reshape(64 * bi_per, 128)
        for k in range(bi_per // 2):
            xk = x64[128 * k : 128 * (k + 1), :]
            xkt = jax.lax.dot_general(
                xk, ident, (((0,), (0,)), ((), ())),
                preferred_element_type=jnp.float32,
                precision=jax.lax.Precision.HIGHEST,
            )
            even = jnp.concatenate([xkt[:64, :64], xkt[64:, :64]], axis=1)
            odd = jnp.concatenate([xkt[:64, 64:], xkt[64:, 64:]], axis=1)
            o_ref[0, :, 2 * k] = even.reshape(8, 8, 128)
            o_ref[0, :, 2 * k + 1] = odd.reshape(8, 8, 128)

    return pl.pallas_call(
        tr,
        grid=(n_bi // bi_per, l_dim),
        in_specs=[
            pl.BlockSpec(
                (8 * bi_per, 8, 128), lambda bi, l: (l * (n_bi // bi_per) + bi, 0, 0)
            )
        ],
        out_specs=pl.BlockSpec(
            (1, 8, bi_per, 8, 128), lambda bi, l: (l, 0, bi, 0, 0)
        ),
        out_shape=jax.ShapeDtypeStruct((l_dim, 8, n_bi, 8, 128), jnp.float32),
    )(g3)


def kernel(token_ids, W):
    b, l = token_ids.shape
    v, d = W.shape
    flat = _gather_lmajor(token_ids.astype(jnp.int32), W)
    g3 = flat.reshape(l * b * d // 1024, 8, 128)
    out5 = _retile_tc(g3, l, NUM_WORKERS)
    y = jnp.transpose(out5, (2, 4, 0, 1, 3))
    return y.reshape(b, l, d)


# speedup vs baseline: 1.0832x; 1.0832x over previous
import functools

import jax
import jax.numpy as jnp
from jax import lax
from jax.experimental import pallas as pl
from jax.experimental.pallas import tpu as pltpu
from jax.experimental.pallas import tpu_sc as plsc

NUM_WORKERS = 32


@jax.jit
def _gather_lmajor(idx, table):
    b, l_dim = idx.shape
    v, d = table.shape
    bpw = b // NUM_WORKERS

    mesh = plsc.VectorSubcoreMesh(core_axis_name="c", subcore_axis_name="s")

    @functools.partial(
        pl.kernel,
        out_type=jax.ShapeDtypeStruct((l_dim * b, d), jnp.float32),
        mesh=mesh,
        scratch_types=[
            pltpu.VMEM((bpw, l_dim), jnp.int32),
            pltpu.VMEM((2, bpw), jnp.int32),
            pltpu.VMEM((2, bpw, d), jnp.float32),
            pltpu.SemaphoreType.DMA,
            pltpu.SemaphoreType.DMA,
            pltpu.SemaphoreType.DMA,
        ],
        compiler_params=pltpu.CompilerParams(
            use_tc_tiling_on_sc=False, needs_layout_passes=False
        ),
    )
    def k(idx_hbm, table_hbm, out_hbm, idx_v, idxcol, bufg, gsem, osem0, osem1):
        wid = lax.axis_index("s") * 2 + lax.axis_index("c")
        b0 = wid * bpw
        pltpu.sync_copy(idx_hbm.at[pl.ds(b0, bpw)], idx_v)
        lanes = lax.iota(jnp.int32, 16)
        osems = (osem0, osem1)

        def build_idxcol(lidx, sel):
            lvec = jnp.full((16,), lidx, jnp.int32)

            def bg(g, c):
                tau = lanes + g * 16
                beta = tau // 2 + (tau % 2) * 64
                idxcol[sel, pl.ds(g * 16, 16)] = plsc.load_gather(idx_v, [beta, lvec])
                return c

            lax.fori_loop(0, bpw // 16, bg, 0)

        def fire_gather(sel):
            pltpu.async_copy(table_hbm.at[idxcol.at[sel]], bufg.at[sel], gsem)

        def wait_gather(sel):
            pltpu.make_async_copy(
                table_hbm.at[idxcol.at[sel]], bufg.at[sel], gsem
            ).wait()

        def out_slice(lidx):
            return out_hbm.at[pl.ds(lidx * b + b0, bpw)]

        def fire_out(lidx, sel):
            pltpu.async_copy(bufg.at[sel], out_slice(lidx), osems[sel])

        def wait_out(lidx, sel):
            pltpu.make_async_copy(bufg.at[sel], out_slice(lidx), osems[sel]).wait()

        def step(lidx, sel, fire_next, wait_o):
            if fire_next:
                build_idxcol(lidx + 1, 1 - sel)
                fire_gather(1 - sel)
            wait_gather(sel)
            if wait_o:
                wait_out(lidx - 2, sel)
            fire_out(lidx, sel)

        build_idxcol(0, 0)
        fire_gather(0)
        step(0, 0, True, False)
        step(1, 1, True, False)

        def body(i, c):
            step(2 * i + 2, 0, True, True)
            step(2 * i + 3, 1, True, True)
            return c

        lax.fori_loop(0, (l_dim - 4) // 2, body, 0)
        step(l_dim - 2, 0, True, True)
        step(l_dim - 1, 1, False, True)
        wait_out(l_dim - 2, 0)
        wait_out(l_dim - 1, 1)

    return k(idx, table)


def _retile_tc(g3, l_dim, n_bi):
    bi_per = 16

    def tr(x_ref, o_ref):
        x64 = x_ref[...].reshape(64 * bi_per, 128)
        for k in range(bi_per // 2):
            xk = x64[128 * k : 128 * (k + 1), :]
            xkt = xk.T
            even = jnp.concatenate([xkt[:64, :64], xkt[64:, :64]], axis=1)
            odd = jnp.concatenate([xkt[:64, 64:], xkt[64:, 64:]], axis=1)
            o_ref[0, :, 2 * k] = even.reshape(8, 8, 128)
            o_ref[0, :, 2 * k + 1] = odd.reshape(8, 8, 128)

    return pl.pallas_call(
        tr,
        grid=(n_bi // bi_per, l_dim),
        in_specs=[
            pl.BlockSpec(
                (8 * bi_per, 8, 128), lambda bi, l: (l * (n_bi // bi_per) + bi, 0, 0)
            )
        ],
        out_specs=pl.BlockSpec(
            (1, 8, bi_per, 8, 128), lambda bi, l: (l, 0, bi, 0, 0)
        ),
        out_shape=jax.ShapeDtypeStruct((l_dim, 8, n_bi, 8, 128), jnp.float32),
    )(g3)


def kernel(token_ids, W):
    b, l = token_ids.shape
    v, d = W.shape
    flat = _gather_lmajor(token_ids.astype(jnp.int32), W)
    g3 = flat.reshape(l * b * d // 1024, 8, 128)
    out5 = _retile_tc(g3, l, NUM_WORKERS)
    y = jnp.transpose(out5, (2, 4, 0, 1, 3))
    return y.reshape(b, l, d)


# speedup vs baseline: 1.2386x; 1.1434x over previous
import functools

import jax
import jax.numpy as jnp
from jax import lax
from jax.experimental import pallas as pl
from jax.experimental.pallas import tpu as pltpu
from jax.experimental.pallas import tpu_sc as plsc

NUM_WORKERS = 32


@jax.jit
def _gather_lmajor(idx, table):
    b, l_dim = idx.shape
    v, d = table.shape
    bpw = b // NUM_WORKERS

    mesh = plsc.VectorSubcoreMesh(core_axis_name="c", subcore_axis_name="s")

    @functools.partial(
        pl.kernel,
        out_type=jax.ShapeDtypeStruct((l_dim * b, d), jnp.float32),
        mesh=mesh,
        scratch_types=[
            pltpu.VMEM((bpw, l_dim), jnp.int32),
            pltpu.VMEM((2, bpw), jnp.int32),
            pltpu.VMEM((2, bpw, d), jnp.float32),
            pltpu.SemaphoreType.DMA,
            pltpu.SemaphoreType.DMA,
            pltpu.SemaphoreType.DMA,
        ],
        compiler_params=pltpu.CompilerParams(
            use_tc_tiling_on_sc=False, needs_layout_passes=False
        ),
    )
    def k(idx_hbm, table_hbm, out_hbm, idx_v, idxcol, bufg, gsem, osem0, osem1):
        wid = lax.axis_index("s") * 2 + lax.axis_index("c")
        b0 = wid * bpw
        pltpu.sync_copy(idx_hbm.at[pl.ds(b0, bpw)], idx_v)
        lanes = lax.iota(jnp.int32, 16)
        osems = (osem0, osem1)

        def build_idxcol(lidx, sel):
            lvec = jnp.full((16,), lidx, jnp.int32)

            def bg(g, c):
                tau = lanes + g * 16
                beta = tau // 2 + (tau % 2) * 64
                idxcol[sel, pl.ds(g * 16, 16)] = plsc.load_gather(idx_v, [beta, lvec])
                return c

            lax.fori_loop(0, bpw // 16, bg, 0)

        def fire_gather(sel):
            pltpu.async_copy(table_hbm.at[idxcol.at[sel]], bufg.at[sel], gsem)

        def wait_gather(sel):
            pltpu.make_async_copy(
                table_hbm.at[idxcol.at[sel]], bufg.at[sel], gsem
            ).wait()

        def out_slice(lidx):
            return out_hbm.at[pl.ds(lidx * b + b0, bpw)]

        def fire_out(lidx, sel):
            pltpu.async_copy(bufg.at[sel], out_slice(lidx), osems[sel])

        def wait_out(lidx, sel):
            pltpu.make_async_copy(bufg.at[sel], out_slice(lidx), osems[sel]).wait()

        def step(lidx, sel, fire_next, wait_o):
            if fire_next:
                build_idxcol(lidx + 1, 1 - sel)
                fire_gather(1 - sel)
            wait_gather(sel)
            if wait_o:
                wait_out(lidx - 2, sel)
            fire_out(lidx, sel)

        build_idxcol(0, 0)
        fire_gather(0)
        step(0, 0, True, False)
        step(1, 1, True, False)

        def body(i, c):
            step(2 * i + 2, 0, True, True)
            step(2 * i + 3, 1, True, True)
            return c

        lax.fori_loop(0, (l_dim - 4) // 2, body, 0)
        step(l_dim - 2, 0, True, True)
        step(l_dim - 1, 1, False, True)
        wait_out(l_dim - 2, 0)
        wait_out(l_dim - 1, 1)

    return k(idx, table)


def _retile_tc(g3, l_dim, n_bi):
    bi_per = 32

    def tr(x_ref, o_ref):
        x64 = x_ref[...].reshape(64 * bi_per, 128)
        for k in range(bi_per // 2):
            xk = x64[128 * k : 128 * (k + 1), :]
            xkt = xk.T
            even = jnp.concatenate([xkt[:64, :64], xkt[64:, :64]], axis=1)
            odd = jnp.concatenate([xkt[:64, 64:], xkt[64:, 64:]], axis=1)
            o_ref[0, :, 2 * k] = even.reshape(8, 8, 128)
            o_ref[0, :, 2 * k + 1] = odd.reshape(8, 8, 128)

    return pl.pallas_call(
        tr,
        grid=(n_bi // bi_per, l_dim),
        in_specs=[
            pl.BlockSpec(
                (8 * bi_per, 8, 128), lambda bi, l: (l * (n_bi // bi_per) + bi, 0, 0)
            )
        ],
        out_specs=pl.BlockSpec(
            (1, 8, bi_per, 8, 128), lambda bi, l: (l, 0, bi, 0, 0)
        ),
        out_shape=jax.ShapeDtypeStruct((l_dim, 8, n_bi, 8, 128), jnp.float32),
    )(g3)


def kernel(token_ids, W):
    b, l = token_ids.shape
    v, d = W.shape
    flat = _gather_lmajor(token_ids.astype(jnp.int32), W)
    g3 = flat.reshape(l * b * d // 1024, 8, 128)
    out5 = _retile_tc(g3, l, NUM_WORKERS)
    y = jnp.transpose(out5, (2, 4, 0, 1, 3))
    return y.reshape(b, l, d)


# speedup vs baseline: 1.2934x; 1.0443x over previous
import functools

import jax
import jax.numpy as jnp
from jax import lax
from jax.experimental import pallas as pl
from jax.experimental.pallas import tpu as pltpu
from jax.experimental.pallas import tpu_sc as plsc

NUM_WORKERS = 32


def _gather_lmajor(idx, table, l_lo, n_l):
    b, l_dim = idx.shape
    v, d = table.shape
    bpw = b // NUM_WORKERS

    mesh = plsc.VectorSubcoreMesh(core_axis_name="c", subcore_axis_name="s")

    @functools.partial(
        pl.kernel,
        out_type=jax.ShapeDtypeStruct((n_l * b, d), jnp.float32),
        mesh=mesh,
        scratch_types=[
            pltpu.VMEM((bpw, l_dim), jnp.int32),
            pltpu.VMEM((2, bpw), jnp.int32),
            pltpu.VMEM((2, bpw, d), jnp.float32),
            pltpu.SemaphoreType.DMA,
            pltpu.SemaphoreType.DMA,
            pltpu.SemaphoreType.DMA,
        ],
        compiler_params=pltpu.CompilerParams(
            use_tc_tiling_on_sc=False, needs_layout_passes=False
        ),
    )
    def k(idx_hbm, table_hbm, out_hbm, idx_v, idxcol, bufg, gsem, osem0, osem1):
        wid = lax.axis_index("s") * 2 + lax.axis_index("c")
        b0 = wid * bpw
        pltpu.sync_copy(idx_hbm.at[pl.ds(b0, bpw)], idx_v)
        lanes = lax.iota(jnp.int32, 16)
        osems = (osem0, osem1)

        def build_idxcol(lidx, sel):
            lvec = jnp.full((16,), l_lo + lidx, jnp.int32)

            def bg(g, c):
                tau = lanes + g * 16
                beta = tau // 2 + (tau % 2) * 64
                idxcol[sel, pl.ds(g * 16, 16)] = plsc.load_gather(idx_v, [beta, lvec])
                return c

            lax.fori_loop(0, bpw // 16, bg, 0)

        def fire_gather(sel):
            pltpu.async_copy(table_hbm.at[idxcol.at[sel]], bufg.at[sel], gsem)

        def wait_gather(sel):
            pltpu.make_async_copy(
                table_hbm.at[idxcol.at[sel]], bufg.at[sel], gsem
            ).wait()

        def out_slice(lidx):
            return out_hbm.at[pl.ds(lidx * b + b0, bpw)]

        def fire_out(lidx, sel):
            pltpu.async_copy(bufg.at[sel], out_slice(lidx), osems[sel])

        def wait_out(lidx, sel):
            pltpu.make_async_copy(bufg.at[sel], out_slice(lidx), osems[sel]).wait()

        def step(lidx, sel, fire_next, wait_o):
            if fire_next:
                build_idxcol(lidx + 1, 1 - sel)
                fire_gather(1 - sel)
            wait_gather(sel)
            if wait_o:
                wait_out(lidx - 2, sel)
            fire_out(lidx, sel)

        build_idxcol(0, 0)
        fire_gather(0)
        step(0, 0, True, False)
        step(1, 1, True, False)

        def body(i, c):
            step(2 * i + 2, 0, True, True)
            step(2 * i + 3, 1, True, True)
            return c

        lax.fori_loop(0, (n_l - 4) // 2, body, 0)
        step(n_l - 2, 0, True, True)
        step(n_l - 1, 1, False, True)
        wait_out(n_l - 2, 0)
        wait_out(n_l - 1, 1)

    return k(idx, table)


def _retile_tc(g3, l_total, n_l, l_off, n_bi, acc=None):
    bi_per = 32

    def tr(*refs):
        x_ref, o_ref = refs[0], refs[-1]
        x64 = x_ref[...].reshape(64 * bi_per, 128)
        for k in range(bi_per // 2):
            xk = x64[128 * k : 128 * (k + 1), :]
            xkt = xk.T
            even = jnp.concatenate([xkt[:64, :64], xkt[64:, :64]], axis=1)
            odd = jnp.concatenate([xkt[:64, 64:], xkt[64:, 64:]], axis=1)
            o_ref[0, :, 2 * k] = even.reshape(8, 8, 128)
            o_ref[0, :, 2 * k + 1] = odd.reshape(8, 8, 128)

    out_shape = jax.ShapeDtypeStruct((l_total, 8, n_bi, 8, 128), jnp.float32)
    in_specs = [
        pl.BlockSpec(
            (8 * bi_per, 8, 128), lambda bi, l: (l * (n_bi // bi_per) + bi, 0, 0)
        )
    ]
    args = [g3]
    aliases = {}
    if acc is not None:
        in_specs.append(pl.BlockSpec(memory_space=pl.ANY))
        args.append(acc)
        aliases = {1: 0}
    return pl.pallas_call(
        tr,
        grid=(n_bi // bi_per, n_l),
        in_specs=in_specs,
        out_specs=pl.BlockSpec(
            (1, 8, bi_per, 8, 128), lambda bi, l: (l + l_off, 0, bi, 0, 0)
        ),
        out_shape=out_shape,
        input_output_aliases=aliases,
    )(*args)


def kernel(token_ids, W):
    b, l = token_ids.shape
    v, d = W.shape
    idx = token_ids.astype(jnp.int32)
    n1 = (l // 2 + 1) // 2 * 2
    n2 = l - n1
    flat1 = _gather_lmajor(idx, W, 0, n1)
    flat2 = _gather_lmajor(idx, W, n1, n2)
    g3_1 = flat1.reshape(n1 * b * d // 1024, 8, 128)
    g3_2 = flat2.reshape(n2 * b * d // 1024, 8, 128)
    part = _retile_tc(g3_1, l, n1, 0, NUM_WORKERS)
    out5 = _retile_tc(g3_2, l, n2, n1, NUM_WORKERS, acc=part)
    y = jnp.transpose(out5, (2, 4, 0, 1, 3))
    return y.reshape(b, l, d)


# speedup vs baseline: 1.3850x; 1.0708x over previous
import functools

import jax
import jax.numpy as jnp
from jax import lax
from jax.experimental import pallas as pl
from jax.experimental.pallas import tpu as pltpu
from jax.experimental.pallas import tpu_sc as plsc

NUM_WORKERS = 32


def _gather_lmajor(idx, table, l_lo, n_l):
    b, l_dim = idx.shape
    v, d = table.shape
    bpw = b // NUM_WORKERS

    mesh = plsc.VectorSubcoreMesh(core_axis_name="c", subcore_axis_name="s")

    @functools.partial(
        pl.kernel,
        out_type=jax.ShapeDtypeStruct((n_l * b, d), jnp.float32),
        mesh=mesh,
        scratch_types=[
            pltpu.VMEM((bpw, l_dim), jnp.int32),
            pltpu.VMEM((2, bpw), jnp.int32),
            pltpu.VMEM((2, bpw, d), jnp.float32),
            pltpu.SemaphoreType.DMA,
            pltpu.SemaphoreType.DMA,
            pltpu.SemaphoreType.DMA,
        ],
        compiler_params=pltpu.CompilerParams(
            use_tc_tiling_on_sc=False, needs_layout_passes=False
        ),
    )
    def k(idx_hbm, table_hbm, out_hbm, idx_v, idxcol, bufg, gsem, osem0, osem1):
        wid = lax.axis_index("s") * 2 + lax.axis_index("c")
        b0 = wid * bpw
        pltpu.sync_copy(idx_hbm.at[pl.ds(b0, bpw)], idx_v)
        lanes = lax.iota(jnp.int32, 16)
        osems = (osem0, osem1)

        def build_idxcol(lidx, sel):
            lvec = jnp.full((16,), l_lo + lidx, jnp.int32)

            def bg(g, c):
                tau = lanes + g * 16
                beta = tau // 2 + (tau % 2) * 64
                idxcol[sel, pl.ds(g * 16, 16)] = plsc.load_gather(idx_v, [beta, lvec])
                return c

            lax.fori_loop(0, bpw // 16, bg, 0)

        def fire_gather(sel):
            pltpu.async_copy(table_hbm.at[idxcol.at[sel]], bufg.at[sel], gsem)

        def wait_gather(sel):
            pltpu.make_async_copy(
                table_hbm.at[idxcol.at[sel]], bufg.at[sel], gsem
            ).wait()

        def out_slice(lidx):
            return out_hbm.at[pl.ds(lidx * b + b0, bpw)]

        def fire_out(lidx, sel):
            pltpu.async_copy(bufg.at[sel], out_slice(lidx), osems[sel])

        def wait_out(lidx, sel):
            pltpu.make_async_copy(bufg.at[sel], out_slice(lidx), osems[sel]).wait()

        def step(lidx, sel, fire_next, wait_o):
            if fire_next:
                build_idxcol(lidx + 1, 1 - sel)
                fire_gather(1 - sel)
            wait_gather(sel)
            if wait_o:
                wait_out(lidx - 2, sel)
            fire_out(lidx, sel)

        build_idxcol(0, 0)
        fire_gather(0)
        step(0, 0, True, False)
        step(1, 1, True, False)

        def body(i, c):
            step(2 * i + 2, 0, True, True)
            step(2 * i + 3, 1, True, True)
            return c

        lax.fori_loop(0, (n_l - 4) // 2, body, 0)
        step(n_l - 2, 0, True, True)
        step(n_l - 1, 1, False, True)
        wait_out(n_l - 2, 0)
        wait_out(n_l - 1, 1)

    return k(idx, table)


def _retile_tc(g3, l_total, n_l, l_off, n_bi, acc=None):
    bi_per = 32
    lp = 2

    def tr(*refs):
        x_ref, o_ref = refs[0], refs[-1]
        for half in range(lp):
            x64 = x_ref[8 * bi_per * half : 8 * bi_per * (half + 1)].reshape(
                64 * bi_per, 128
            )
            for k in range(bi_per // 2):
                xk = x64[128 * k : 128 * (k + 1), :]
                xkt = xk.T
                even = jnp.concatenate([xkt[:64, :64], xkt[64:, :64]], axis=1)
                odd = jnp.concatenate([xkt[:64, 64:], xkt[64:, 64:]], axis=1)
                o_ref[half, :, 2 * k] = even.reshape(8, 8, 128)
                o_ref[half, :, 2 * k + 1] = odd.reshape(8, 8, 128)

    out_shape = jax.ShapeDtypeStruct((l_total, 8, n_bi, 8, 128), jnp.float32)
    in_specs = [
        pl.BlockSpec((8 * bi_per * lp, 8, 128), lambda l: (l, 0, 0))
    ]
    args = [g3]
    aliases = {}
    if acc is not None:
        in_specs.append(pl.BlockSpec(memory_space=pl.ANY))
        args.append(acc)
        aliases = {1: 0}
    return pl.pallas_call(
        tr,
        grid=(n_l // lp,),
        in_specs=in_specs,
        out_specs=pl.BlockSpec(
            (lp, 8, bi_per, 8, 128), lambda l: (l + l_off // lp, 0, 0, 0, 0)
        ),
        out_shape=out_shape,
        input_output_aliases=aliases,
    )(*args)


def kernel(token_ids, W):
    b, l = token_ids.shape
    v, d = W.shape
    idx = token_ids.astype(jnp.int32)
    n1 = (l // 2 + 1) // 2 * 2
    n2 = l - n1
    flat1 = _gather_lmajor(idx, W, 0, n1)
    flat2 = _gather_lmajor(idx, W, n1, n2)
    g3_1 = flat1.reshape(n1 * b * d // 1024, 8, 128)
    g3_2 = flat2.reshape(n2 * b * d // 1024, 8, 128)
    part = _retile_tc(g3_1, l, n1, 0, NUM_WORKERS)
    out5 = _retile_tc(g3_2, l, n2, n1, NUM_WORKERS, acc=part)
    y = jnp.transpose(out5, (2, 4, 0, 1, 3))
    return y.reshape(b, l, d)
